# baseline (speedup 1.0000x reference)
import functools

import jax
import jax.numpy as jnp
from jax import lax
from jax.experimental import pallas as pl
from jax.experimental.pallas import tpu as pltpu
from jax.experimental.pallas import tpu_sc as plsc

SHAPE = (4096, 4096)
NUMEL = SHAPE[0] * SHAPE[1]
K = 1677721

NC = 2
NS = 16
NW = NC * NS

CH = 16384
ROWS_PER_CHUNK = CH // SHAPE[1]
NCHUNK = NUMEL // CH
ROUNDS = NCHUNK // NW
NBUF = 6
NFLIGHT = 3
VB = 2048
K_MAIN = K - (K % 8)
BS_ITERS = K.bit_length()
COL_BITS = SHAPE[1].bit_length() - 1
COL_MASK = SHAPE[1] - 1


def _sc_body(tensor_hbm, values_hbm, indices_hbm, tailix_hbm, tailval_hbm,
             out_hbm, acc0, acc1, acc2, acc3, acc4, acc5, idxb, valb,
             stv0, stv1, stv2, midb0, gatb0, midb1, gatb1, midb2, gatb2,
             tiv, tvv, is0, is1, is2, is3, is4, is5,
             os0, os1, os2, os3, os4, os5, msem):
    accs = (acc0, acc1, acc2, acc3, acc4, acc5)
    isems = (is0, is1, is2, is3, is4, is5)
    osems = (os0, os1, os2, os3, os4, os5)
    stvs = (stv0, stv1, stv2)
    midbs = (midb0, midb1, midb2)
    gatbs = (gatb0, gatb1, gatb2)

    def _copy(src_ref, dst_ref):
        pltpu.async_copy(src_ref, dst_ref, msem).wait()

    cidx = lax.axis_index("c")
    sidx = lax.axis_index("s")
    wid = sidx * NC + cidx
    lanes = lax.iota(jnp.int32, 16)

    def chunk_row0(r):
        return pl.multiple_of((wid * ROUNDS + r) * ROWS_PER_CHUNK,
                              ROWS_PER_CHUNK)

    def start_in(r):
        pltpu.make_async_copy(
            tensor_hbm.at[pl.ds(chunk_row0(r), ROWS_PER_CHUNK), :],
            accs[r % NBUF], isems[r % NBUF]).start()

    def wait_in(r):
        pltpu.make_async_copy(
            tensor_hbm.at[pl.ds(chunk_row0(r), ROWS_PER_CHUNK), :],
            accs[r % NBUF], isems[r % NBUF]).wait()

    def start_out(r):
        pltpu.make_async_copy(
            accs[r % NBUF],
            out_hbm.at[pl.ds(chunk_row0(r), ROWS_PER_CHUNK), :],
            osems[r % NBUF]).start()

    def wait_out(r):
        pltpu.make_async_copy(
            accs[r % NBUF],
            out_hbm.at[pl.ds(chunk_row0(r), ROWS_PER_CHUNK), :],
            osems[r % NBUF]).wait()

    for r in range(NFLIGHT):
        start_in(r)

    _copy(tailix_hbm, tiv)
    _copy(tailval_hbm, tvv)
    tail_i = tiv[...]
    tail_v = tvv[...]

    targets = [
        jnp.minimum((wid * ROUNDS + (q * 16) + lanes), NCHUNK) * CH
        for q in range(3)
    ]

    def bs_body(it, carry):
        lohs = [carry[0:2], carry[2:4], carry[4:6]]
        mids = []
        for q in range(3):
            lo_v, hi_v = lohs[q]
            mid = lo_v + ((hi_v - lo_v) >> 1)
            midbs[q][...] = jnp.minimum(mid, K - 1)
            mids.append(mid)
        cps = [pltpu.async_copy(indices_hbm.at[midbs[q]], gatbs[q], msem)
               for q in range(3)]
        out = []
        for q in range(3):
            cps[q].wait()
        for q in range(3):
            lo_v, hi_v = lohs[q]
            active = lo_v < hi_v
            gr = gatbs[q][...] < targets[q]
            out.append(jnp.where(active & gr, mids[q] + 1, lo_v))
            out.append(jnp.where(active & (~gr), mids[q], hi_v))
        return tuple(out)

    z16 = jnp.zeros((16,), jnp.int32)
    k16 = jnp.full((16,), K, jnp.int32)
    res = lax.fori_loop(0, BS_ITERS, bs_body, (z16, k16, z16, k16, z16, k16))
    for q in range(3):
        stvs[q][...] = res[2 * q]
    svs = [stvs[q][...] for q in range(3)]

    def bound(i):
        return svs[i // 16][i % 16]

    for r in range(ROUNDS):
        acc = accs[r % NBUF]
        chunk = wid * ROUNDS + r
        lo = pl.multiple_of(chunk * CH, CH)
        p0 = bound(r)
        p1 = bound(r + 1)
        p1e = jnp.minimum(p1, K_MAIN)

        wait_in(r)

        sbase0 = pl.multiple_of(
            jnp.maximum(jnp.minimum(p0 & -8, K_MAIN - VB), 0), 8)
        nb = jnp.maximum((p1e - sbase0 + (VB - 1)) // VB, 0)

        def batch_body(i, carry, _p0=p0, _p1e=p1e, _lo=lo, _sbase0=sbase0,
                       _acc=acc):
            ustart = _sbase0 + i * VB
            sbase = pl.multiple_of(jnp.minimum(ustart, K_MAIN - VB), 8)
            vcp = pltpu.async_copy(values_hbm.at[pl.ds(sbase, VB)], valb,
                                   msem)
            icp = pltpu.async_copy(indices_hbm.at[pl.ds(sbase, VB)], idxb,
                                   msem)
            vcp.wait()
            icp.wait()
            cur = jnp.maximum(_p0, ustart)

            def inner(j, carry2):
                off = pl.multiple_of(j * 16, 16)
                iv = idxb[pl.ds(off, 16)]
                vv = valb[pl.ds(off, 16)]
                pos = sbase + j * 16 + lanes
                m = (pos >= cur) & (pos < _p1e)
                liv = jnp.where(m, iv - _lo, 0)
                plsc.addupdate_scatter(
                    _acc, [liv >> COL_BITS, liv & COL_MASK], vv, mask=m)
                return carry2

            lax.fori_loop(0, VB // 16, inner, 0)
            return carry

        lax.fori_loop(0, nb, batch_body, 0)

        tm = (tail_i >= lo) & (tail_i < lo + CH)
        tl = jnp.where(tm, tail_i - lo, 0)
        plsc.addupdate_scatter(acc, [tl >> COL_BITS, tl & COL_MASK], tail_v,
                               mask=tm)

        start_out(r)

        if r + NFLIGHT < ROUNDS:
            prev = r + NFLIGHT - NBUF
            if prev >= 0:
                wait_out(prev)
            start_in(r + NFLIGHT)

    last_waited = (ROUNDS - NFLIGHT - 1) + NFLIGHT - NBUF
    for r in range(max(last_waited + 1, 0), ROUNDS):
        wait_out(r)


_sc_call = functools.partial(
    pl.kernel,
    out_type=jax.ShapeDtypeStruct(SHAPE, jnp.float32),
    mesh=plsc.VectorSubcoreMesh(core_axis_name="c", subcore_axis_name="s"),
    compiler_params=pltpu.CompilerParams(needs_layout_passes=False),
    scratch_types=(
        [pltpu.VMEM((ROWS_PER_CHUNK, SHAPE[1]), jnp.float32)] * NBUF
        + [
            pltpu.VMEM((VB,), jnp.int32),
            pltpu.VMEM((VB,), jnp.float32),
            pltpu.VMEM((16,), jnp.int32),
            pltpu.VMEM((16,), jnp.int32),
            pltpu.VMEM((16,), jnp.int32),
            pltpu.VMEM((16,), jnp.int32),
            pltpu.VMEM((16,), jnp.int32),
            pltpu.VMEM((16,), jnp.int32),
            pltpu.VMEM((16,), jnp.int32),
            pltpu.VMEM((16,), jnp.int32),
            pltpu.VMEM((16,), jnp.int32),
            pltpu.VMEM((16,), jnp.int32),
            pltpu.VMEM((16,), jnp.float32),
        ]
        + [pltpu.SemaphoreType.DMA] * (2 * NBUF)
        + [pltpu.SemaphoreType.DMA]
    ),
)(_sc_body)


def kernel(tensor, values, indices):
    ntail = K - K_MAIN
    tail_i = jnp.full((16,), -1, jnp.int32).at[:ntail].set(indices[K_MAIN:])
    tail_v = jnp.zeros((16,), jnp.float32).at[:ntail].set(values[K_MAIN:])
    return _sc_call(tensor, values, indices, tail_i, tail_v)


# speedup vs baseline: 1.1201x; 1.1201x over previous
import functools

import jax
import jax.numpy as jnp
from jax import lax
from jax.experimental import pallas as pl
from jax.experimental.pallas import tpu as pltpu
from jax.experimental.pallas import tpu_sc as plsc

SHAPE = (4096, 4096)
NUMEL = SHAPE[0] * SHAPE[1]
K = 1677721

NC = 2
NS = 16
NW = NC * NS

CH = 32768
ROWS_PER_CHUNK = CH // SHAPE[1]
NCHUNK = NUMEL // CH
ROUNDS = NCHUNK // NW
NBUF = 3
VB = 4096
K_MAIN = K - (K % 8)
BS_ITERS = K.bit_length()
COL_BITS = SHAPE[1].bit_length() - 1
COL_MASK = SHAPE[1] - 1


def _sc_body(tensor_hbm, values_hbm, indices_hbm, tailix_hbm, tailval_hbm,
             out_hbm, acc0, acc1, acc2, idxb0, valb0, idxb1, valb1,
             stv0, stv1, midb, gatb, midb2, gatb2, tiv, tvv,
             isem0, isem1, isem2, osem0, osem1, osem2, bsem0, bsem1, msem):
    accs = (acc0, acc1, acc2)
    isems = (isem0, isem1, isem2)
    osems = (osem0, osem1, osem2)
    idxbs = (idxb0, idxb1)
    valbs = (valb0, valb1)
    bsems = (bsem0, bsem1)
    stvs = (stv0, stv1)

    def _copy(src_ref, dst_ref):
        pltpu.async_copy(src_ref, dst_ref, msem).wait()

    cidx = lax.axis_index("c")
    sidx = lax.axis_index("s")
    wid = sidx * NC + cidx
    lanes = lax.iota(jnp.int32, 16)

    def chunk_row0(r):
        return pl.multiple_of((wid * ROUNDS + r) * ROWS_PER_CHUNK,
                              ROWS_PER_CHUNK)

    def start_in(r):
        pltpu.make_async_copy(
            tensor_hbm.at[pl.ds(chunk_row0(r), ROWS_PER_CHUNK), :],
            accs[r % NBUF], isems[r % NBUF]).start()

    def wait_in(r):
        pltpu.make_async_copy(
            tensor_hbm.at[pl.ds(chunk_row0(r), ROWS_PER_CHUNK), :],
            accs[r % NBUF], isems[r % NBUF]).wait()

    def start_out(r):
        pltpu.make_async_copy(
            accs[r % NBUF],
            out_hbm.at[pl.ds(chunk_row0(r), ROWS_PER_CHUNK), :],
            osems[r % NBUF]).start()

    def wait_out(r):
        pltpu.make_async_copy(
            accs[r % NBUF],
            out_hbm.at[pl.ds(chunk_row0(r), ROWS_PER_CHUNK), :],
            osems[r % NBUF]).wait()

    for r in range(min(2, ROUNDS)):
        start_in(r)

    _copy(tailix_hbm, tiv)
    _copy(tailval_hbm, tvv)
    tail_i = tiv[...]
    tail_v = tvv[...]

    targets1 = (wid * ROUNDS + lanes) * CH
    targets2 = ((wid + 1) * ROUNDS) * CH + jnp.zeros((16,), jnp.int32)

    def bs_body(it, carry):
        lo1, hi1, lo2, hi2 = carry
        a1 = lo1 < hi1
        a2 = lo2 < hi2
        mid1 = lo1 + ((hi1 - lo1) >> 1)
        mid2 = lo2 + ((hi2 - lo2) >> 1)
        midb[...] = jnp.minimum(mid1, K - 1)
        midb2[...] = jnp.minimum(mid2, K - 1)
        c1 = pltpu.async_copy(indices_hbm.at[midb], gatb, msem)
        c2 = pltpu.async_copy(indices_hbm.at[midb2], gatb2, msem)
        c1.wait()
        c2.wait()
        gr1 = gatb[...] < targets1
        gr2 = gatb2[...] < targets2
        return (jnp.where(a1 & gr1, mid1 + 1, lo1),
                jnp.where(a1 & (~gr1), mid1, hi1),
                jnp.where(a2 & gr2, mid2 + 1, lo2),
                jnp.where(a2 & (~gr2), mid2, hi2))

    z16 = jnp.zeros((16,), jnp.int32)
    k16 = jnp.full((16,), K, jnp.int32)
    lo1_v, _, lo2_v, _ = lax.fori_loop(
        0, BS_ITERS, bs_body, (z16, k16, z16, k16))
    stvs[0][...] = lo1_v
    stvs[1][...] = lo2_v

    sv0 = stvs[0][...]
    sv1 = stvs[1][...]

    def bound(i):
        return sv0[i] if i < 16 else sv1[0]

    def batch_sbase(sbase0, i):
        return pl.multiple_of(
            jnp.minimum(sbase0 + i * VB, K_MAIN - VB), 8)

    def start_batch(sbase0, i, b):
        sbase = batch_sbase(sbase0, i)
        pltpu.make_async_copy(values_hbm.at[pl.ds(sbase, VB)], valbs[b],
                              bsems[b]).start()
        pltpu.make_async_copy(indices_hbm.at[pl.ds(sbase, VB)], idxbs[b],
                              bsems[b]).start()

    def wait_batch(sbase0, i, b):
        sbase = batch_sbase(sbase0, i)
        pltpu.make_async_copy(values_hbm.at[pl.ds(sbase, VB)], valbs[b],
                              bsems[b]).wait()
        pltpu.make_async_copy(indices_hbm.at[pl.ds(sbase, VB)], idxbs[b],
                              bsems[b]).wait()

    for r in range(ROUNDS):
        acc = accs[r % NBUF]
        chunk = wid * ROUNDS + r
        lo = pl.multiple_of(chunk * CH, CH)
        p0 = bound(r)
        p1 = bound(r + 1)
        p1e = jnp.minimum(p1, K_MAIN)

        wait_in(r)

        sbase0 = pl.multiple_of(
            jnp.maximum(jnp.minimum(p0 & -8, K_MAIN - VB), 0), 8)
        nb = jnp.maximum((p1e - sbase0 + (VB - 1)) // VB, 0)

        @pl.when(nb > 0)
        def _():
            start_batch(sbase0, 0, 0)

        @pl.when(nb > 1)
        def _():
            start_batch(sbase0, 1, 1)

        def scatter_batch(i, b, _p0=p0, _p1e=p1e, _lo=lo, _sbase0=sbase0,
                          _acc=acc):
            sbase = batch_sbase(_sbase0, i)
            cur = jnp.maximum(_p0, _sbase0 + i * VB)

            def inner(j, carry2):
                off = pl.multiple_of(j * 16, 16)
                iv = idxbs[b][pl.ds(off, 16)]
                vv = valbs[b][pl.ds(off, 16)]
                pos = sbase + j * 16 + lanes
                m = (pos >= cur) & (pos < _p1e)
                liv = jnp.where(m, iv - _lo, 0)
                plsc.addupdate_scatter(
                    _acc, [liv >> COL_BITS, liv & COL_MASK], vv, mask=m)
                return carry2

            lax.fori_loop(0, VB // 16, inner, 0)

        def pair_body(g, carry, _sbase0=sbase0, _nb=nb, _p0=p0, _p1e=p1e,
                      _lo=lo, _acc=acc):
            i0 = 2 * g
            i1 = 2 * g + 1
            wait_batch(_sbase0, i0, 0)
            scatter_batch(i0, 0, _p0=_p0, _p1e=_p1e, _lo=_lo,
                          _sbase0=_sbase0, _acc=_acc)

            @pl.when(i0 + 2 < _nb)
            def _():
                start_batch(_sbase0, i0 + 2, 0)

            @pl.when(i1 < _nb)
            def _():
                wait_batch(_sbase0, i1, 1)
                scatter_batch(i1, 1, _p0=_p0, _p1e=_p1e, _lo=_lo,
                              _sbase0=_sbase0, _acc=_acc)

                @pl.when(i1 + 2 < _nb)
                def _():
                    start_batch(_sbase0, i1 + 2, 1)

            return carry

        lax.fori_loop(0, (nb + 1) // 2, pair_body, 0)

        tm = (tail_i >= lo) & (tail_i < lo + CH)
        tl = jnp.where(tm, tail_i - lo, 0)
        plsc.addupdate_scatter(acc, [tl >> COL_BITS, tl & COL_MASK], tail_v,
                               mask=tm)

        start_out(r)

        if r + 2 < ROUNDS:
            if r >= 1:
                wait_out(r - 1)
            start_in(r + 2)

    for r in range(max(ROUNDS - NBUF, 0), ROUNDS):
        wait_out(r)


_sc_call = functools.partial(
    pl.kernel,
    out_type=jax.ShapeDtypeStruct(SHAPE, jnp.float32),
    mesh=plsc.VectorSubcoreMesh(core_axis_name="c", subcore_axis_name="s"),
    compiler_params=pltpu.CompilerParams(needs_layout_passes=False),
    scratch_types=(
        [pltpu.VMEM((ROWS_PER_CHUNK, SHAPE[1]), jnp.float32)] * NBUF
        + [
            pltpu.VMEM((VB,), jnp.int32),
            pltpu.VMEM((VB,), jnp.float32),
            pltpu.VMEM((VB,), jnp.int32),
            pltpu.VMEM((VB,), jnp.float32),
            pltpu.VMEM((16,), jnp.int32),
            pltpu.VMEM((16,), jnp.int32),
            pltpu.VMEM((16,), jnp.int32),
            pltpu.VMEM((16,), jnp.int32),
            pltpu.VMEM((16,), jnp.int32),
            pltpu.VMEM((16,), jnp.int32),
            pltpu.VMEM((16,), jnp.int32),
            pltpu.VMEM((16,), jnp.float32),
        ]
        + [pltpu.SemaphoreType.DMA] * (2 * NBUF)
        + [pltpu.SemaphoreType.DMA] * 2
        + [pltpu.SemaphoreType.DMA]
    ),
)(_sc_body)


def kernel(tensor, values, indices):
    ntail = K - K_MAIN
    tail_i = jnp.full((16,), -1, jnp.int32).at[:ntail].set(indices[K_MAIN:])
    tail_v = jnp.zeros((16,), jnp.float32).at[:ntail].set(values[K_MAIN:])
    return _sc_call(tensor, values, indices, tail_i, tail_v)


# speedup vs baseline: 1.1447x; 1.0220x over previous
import functools

import jax
import jax.numpy as jnp
from jax import lax
from jax.experimental import pallas as pl
from jax.experimental.pallas import tpu as pltpu
from jax.experimental.pallas import tpu_sc as plsc

SHAPE = (4096, 4096)
NUMEL = SHAPE[0] * SHAPE[1]
K = 1677721

NC = 2
NS = 16
NW = NC * NS

CH = 32768
ROWS_PER_CHUNK = CH // SHAPE[1]
NCHUNK = NUMEL // CH
ROUNDS = NCHUNK // NW
NBUF = 3
VB = 4096
NTAIL = K % 8
K_MAIN = K - NTAIL
BS_ITERS = K.bit_length()
COL_BITS = SHAPE[1].bit_length() - 1
COL_MASK = SHAPE[1] - 1


def _sc_body(tensor_hbm, values_hbm, indices_hbm, out_hbm,
             acc0, acc1, acc2, idxb, valb, stv0, stv1, midb, gatb,
             midb2, gatb2, tiv, tvv,
             isem0, isem1, isem2, osem0, osem1, osem2, msem):
    accs = (acc0, acc1, acc2)
    isems = (isem0, isem1, isem2)
    osems = (osem0, osem1, osem2)
    stvs = (stv0, stv1)

    cidx = lax.axis_index("c")
    sidx = lax.axis_index("s")
    wid = sidx * NC + cidx
    lanes = lax.iota(jnp.int32, 16)

    def chunk_row0(r):
        return pl.multiple_of((wid * ROUNDS + r) * ROWS_PER_CHUNK,
                              ROWS_PER_CHUNK)

    def start_in(r):
        pltpu.make_async_copy(
            tensor_hbm.at[pl.ds(chunk_row0(r), ROWS_PER_CHUNK), :],
            accs[r % NBUF], isems[r % NBUF]).start()

    def wait_in(r):
        pltpu.make_async_copy(
            tensor_hbm.at[pl.ds(chunk_row0(r), ROWS_PER_CHUNK), :],
            accs[r % NBUF], isems[r % NBUF]).wait()

    def start_out(r):
        pltpu.make_async_copy(
            accs[r % NBUF],
            out_hbm.at[pl.ds(chunk_row0(r), ROWS_PER_CHUNK), :],
            osems[r % NBUF]).start()

    def wait_out(r):
        pltpu.make_async_copy(
            accs[r % NBUF],
            out_hbm.at[pl.ds(chunk_row0(r), ROWS_PER_CHUNK), :],
            osems[r % NBUF]).wait()

    for r in range(min(2, ROUNDS)):
        start_in(r)

    tiv[...] = jnp.full((16,), -1, jnp.int32)
    tvv[...] = jnp.zeros((16,), jnp.float32)
    if NTAIL:
        tic = pltpu.async_copy(
            indices_hbm.at[pl.ds(K_MAIN, NTAIL)],
            tiv.at[pl.ds(0, NTAIL)], msem)
        tvc = pltpu.async_copy(
            values_hbm.at[pl.ds(K_MAIN, NTAIL)],
            tvv.at[pl.ds(0, NTAIL)], msem)
        tic.wait()
        tvc.wait()
    tail_i = tiv[...]
    tail_v = tvv[...]

    targets1 = (wid * ROUNDS + lanes) * CH
    targets2 = ((wid + 1) * ROUNDS) * CH + jnp.zeros((16,), jnp.int32)

    def bs_body(it, carry):
        lo1, hi1, lo2, hi2 = carry
        a1 = lo1 < hi1
        a2 = lo2 < hi2
        mid1 = lo1 + ((hi1 - lo1) >> 1)
        mid2 = lo2 + ((hi2 - lo2) >> 1)
        midb[...] = jnp.minimum(mid1, K - 1)
        midb2[...] = jnp.minimum(mid2, K - 1)
        c1 = pltpu.async_copy(indices_hbm.at[midb], gatb, msem)
        c2 = pltpu.async_copy(indices_hbm.at[midb2], gatb2, msem)
        c1.wait()
        c2.wait()
        gr1 = gatb[...] < targets1
        gr2 = gatb2[...] < targets2
        return (jnp.where(a1 & gr1, mid1 + 1, lo1),
                jnp.where(a1 & (~gr1), mid1, hi1),
                jnp.where(a2 & gr2, mid2 + 1, lo2),
                jnp.where(a2 & (~gr2), mid2, hi2))

    z16 = jnp.zeros((16,), jnp.int32)
    k16 = jnp.full((16,), K, jnp.int32)
    lo1_v, _, lo2_v, _ = lax.fori_loop(
        0, BS_ITERS, bs_body, (z16, k16, z16, k16))
    stvs[0][...] = lo1_v
    stvs[1][...] = lo2_v

    sv0 = stvs[0][...]
    sv1 = stvs[1][...]

    def bound(i):
        return sv0[i] if i < 16 else sv1[0]

    for r in range(ROUNDS):
        acc = accs[r % NBUF]
        chunk = wid * ROUNDS + r
        lo = pl.multiple_of(chunk * CH, CH)
        p0 = bound(r)
        p1 = bound(r + 1)
        p1e = jnp.minimum(p1, K_MAIN)

        wait_in(r)

        sbase0 = pl.multiple_of(
            jnp.maximum(jnp.minimum(p0 & -8, K_MAIN - VB), 0), 8)
        nb = jnp.maximum((p1e - sbase0 + (VB - 1)) // VB, 0)

        def batch_body(i, carry, _p0=p0, _p1e=p1e, _lo=lo, _sbase0=sbase0,
                       _acc=acc):
            ustart = _sbase0 + i * VB
            sbase = pl.multiple_of(jnp.minimum(ustart, K_MAIN - VB), 8)
            vcp = pltpu.async_copy(values_hbm.at[pl.ds(sbase, VB)], valb,
                                   msem)
            icp = pltpu.async_copy(indices_hbm.at[pl.ds(sbase, VB)], idxb,
                                   msem)
            vcp.wait()
            icp.wait()
            cur = jnp.maximum(_p0, ustart)

            def inner(j, carry2):
                off = pl.multiple_of(j * 16, 16)
                iv = idxb[pl.ds(off, 16)]
                vv = valb[pl.ds(off, 16)]
                pos = sbase + j * 16 + lanes
                m = (pos >= cur) & (pos < _p1e)
                liv = jnp.where(m, iv - _lo, 0)
                plsc.addupdate_scatter(
                    _acc, [liv >> COL_BITS, liv & COL_MASK], vv, mask=m)
                return carry2

            lax.fori_loop(0, VB // 16, inner, 0)
            return carry

        lax.fori_loop(0, nb, batch_body, 0)

        tm = (tail_i >= lo) & (tail_i < lo + CH)
        tl = jnp.where(tm, tail_i - lo, 0)
        plsc.addupdate_scatter(acc, [tl >> COL_BITS, tl & COL_MASK], tail_v,
                               mask=tm)

        start_out(r)

        if r + 2 < ROUNDS:
            if r >= 1:
                wait_out(r - 1)
            start_in(r + 2)

    for r in range(max(ROUNDS - NBUF, 0), ROUNDS):
        wait_out(r)


_sc_call = functools.partial(
    pl.kernel,
    out_type=jax.ShapeDtypeStruct(SHAPE, jnp.float32),
    mesh=plsc.VectorSubcoreMesh(core_axis_name="c", subcore_axis_name="s"),
    compiler_params=pltpu.CompilerParams(needs_layout_passes=False),
    scratch_types=(
        [pltpu.VMEM((ROWS_PER_CHUNK, SHAPE[1]), jnp.float32)] * NBUF
        + [
            pltpu.VMEM((VB,), jnp.int32),
            pltpu.VMEM((VB,), jnp.float32),
            pltpu.VMEM((16,), jnp.int32),
            pltpu.VMEM((16,), jnp.int32),
            pltpu.VMEM((16,), jnp.int32),
            pltpu.VMEM((16,), jnp.int32),
            pltpu.VMEM((16,), jnp.int32),
            pltpu.VMEM((16,), jnp.int32),
            pltpu.VMEM((16,), jnp.int32),
            pltpu.VMEM((16,), jnp.float32),
        ]
        + [pltpu.SemaphoreType.DMA] * (2 * NBUF)
        + [pltpu.SemaphoreType.DMA]
    ),
)(_sc_body)


def kernel(tensor, values, indices):
    return _sc_call(tensor, values, indices)


# speedup vs baseline: 1.2241x; 1.0694x over previous
import functools

import jax
import jax.numpy as jnp
from jax import lax
from jax.experimental import pallas as pl
from jax.experimental.pallas import tpu as pltpu
from jax.experimental.pallas import tpu_sc as plsc

SHAPE = (4096, 4096)
NUMEL = SHAPE[0] * SHAPE[1]
K = 1677721

NC = 2
NS = 16
NW = NC * NS

CH = 32768
ROWS_PER_CHUNK = CH // SHAPE[1]
NCHUNK = NUMEL // CH
ROUNDS = NCHUNK // NW
NBUF = 3
VB = 4096
NTAIL = K % 8
K_MAIN = K - NTAIL
BS_ITERS = K.bit_length()
COL_BITS = SHAPE[1].bit_length() - 1
COL_MASK = SHAPE[1] - 1


def _sc_body(tensor_hbm, values_hbm, indices_hbm, out_hbm,
             acc0, acc1, acc2, idxb, valb, stv0, stv1, midb, gatb,
             midb2, gatb2, tiv, tvv,
             isem0, isem1, isem2, osem0, osem1, osem2, msem):
    accs = (acc0, acc1, acc2)
    isems = (isem0, isem1, isem2)
    osems = (osem0, osem1, osem2)
    stvs = (stv0, stv1)

    cidx = lax.axis_index("c")
    sidx = lax.axis_index("s")
    wid = sidx * NC + cidx
    lanes = lax.iota(jnp.int32, 16)

    def chunk_row0(r):
        return pl.multiple_of((wid * ROUNDS + r) * ROWS_PER_CHUNK,
                              ROWS_PER_CHUNK)

    def start_in(r):
        pltpu.make_async_copy(
            tensor_hbm.at[pl.ds(chunk_row0(r), ROWS_PER_CHUNK), :],
            accs[r % NBUF], isems[r % NBUF]).start()

    def wait_in(r):
        pltpu.make_async_copy(
            tensor_hbm.at[pl.ds(chunk_row0(r), ROWS_PER_CHUNK), :],
            accs[r % NBUF], isems[r % NBUF]).wait()

    def start_out(r):
        pltpu.make_async_copy(
            accs[r % NBUF],
            out_hbm.at[pl.ds(chunk_row0(r), ROWS_PER_CHUNK), :],
            osems[r % NBUF]).start()

    def wait_out(r):
        pltpu.make_async_copy(
            accs[r % NBUF],
            out_hbm.at[pl.ds(chunk_row0(r), ROWS_PER_CHUNK), :],
            osems[r % NBUF]).wait()

    for r in range(min(2, ROUNDS)):
        start_in(r)

    tiv[...] = jnp.full((16,), -1, jnp.int32)
    tvv[...] = jnp.zeros((16,), jnp.float32)
    if NTAIL:
        tic = pltpu.async_copy(
            indices_hbm.at[pl.ds(K_MAIN, NTAIL)],
            tiv.at[pl.ds(0, NTAIL)], msem)
        tvc = pltpu.async_copy(
            values_hbm.at[pl.ds(K_MAIN, NTAIL)],
            tvv.at[pl.ds(0, NTAIL)], msem)
        tic.wait()
        tvc.wait()
    tail_i = tiv[...]
    tail_v = tvv[...]

    targets1 = (wid * ROUNDS + lanes) * CH
    targets2 = ((wid + 1) * ROUNDS) * CH + jnp.zeros((16,), jnp.int32)

    def bs_body(it, carry):
        lo1, hi1, lo2, hi2 = carry
        a1 = lo1 < hi1
        a2 = lo2 < hi2
        mid1 = lo1 + ((hi1 - lo1) >> 1)
        mid2 = lo2 + ((hi2 - lo2) >> 1)
        midb[...] = jnp.minimum(mid1, K - 1)
        midb2[...] = jnp.minimum(mid2, K - 1)
        c1 = pltpu.async_copy(indices_hbm.at[midb], gatb, msem)
        c2 = pltpu.async_copy(indices_hbm.at[midb2], gatb2, msem)
        c1.wait()
        c2.wait()
        gr1 = gatb[...] < targets1
        gr2 = gatb2[...] < targets2
        return (jnp.where(a1 & gr1, mid1 + 1, lo1),
                jnp.where(a1 & (~gr1), mid1, hi1),
                jnp.where(a2 & gr2, mid2 + 1, lo2),
                jnp.where(a2 & (~gr2), mid2, hi2))

    z16 = jnp.zeros((16,), jnp.int32)
    k16 = jnp.full((16,), K, jnp.int32)
    lo1_v, _, lo2_v, _ = lax.fori_loop(
        0, BS_ITERS, bs_body, (z16, k16, z16, k16))
    stvs[0][...] = lo1_v
    stvs[1][...] = lo2_v

    sv0 = stvs[0][...]
    sv1 = stvs[1][...]

    def bound(i):
        return sv0[i] if i < 16 else sv1[0]

    for r in range(ROUNDS):
        acc = accs[r % NBUF]
        chunk = wid * ROUNDS + r
        lo = pl.multiple_of(chunk * CH, CH)
        p0 = bound(r)
        p1 = bound(r + 1)
        p1e = jnp.minimum(p1, K_MAIN)

        wait_in(r)

        sbase0 = pl.multiple_of(
            jnp.maximum(jnp.minimum(p0 & -8, K_MAIN - VB), 0), 8)
        nb = jnp.maximum((p1e - sbase0 + (VB - 1)) // VB, 0)

        def batch_body(i, carry, _p0=p0, _p1e=p1e, _lo=lo, _sbase0=sbase0,
                       _acc=acc):
            ustart = _sbase0 + i * VB
            sbase = pl.multiple_of(jnp.minimum(ustart, K_MAIN - VB), 8)
            vcp = pltpu.async_copy(values_hbm.at[pl.ds(sbase, VB)], valb,
                                   msem)
            icp = pltpu.async_copy(indices_hbm.at[pl.ds(sbase, VB)], idxb,
                                   msem)
            vcp.wait()
            icp.wait()
            cur = jnp.maximum(_p0, ustart)
            j0 = (cur - sbase) >> 4
            jend = (jnp.minimum(_p1e, sbase + VB) - sbase + 15) >> 4

            def inner(j, carry2):
                off = pl.multiple_of(j * 16, 16)
                iv = idxb[pl.ds(off, 16)]
                vv = valb[pl.ds(off, 16)]
                pos = sbase + j * 16 + lanes
                m = (pos >= cur) & (pos < _p1e)
                liv = jnp.where(m, iv - _lo, 0)
                plsc.addupdate_scatter(
                    _acc, [liv >> COL_BITS, liv & COL_MASK], vv, mask=m)
                return carry2

            lax.fori_loop(j0, jend, inner, 0)
            return carry

        lax.fori_loop(0, nb, batch_body, 0)

        tm = (tail_i >= lo) & (tail_i < lo + CH)
        tl = jnp.where(tm, tail_i - lo, 0)
        plsc.addupdate_scatter(acc, [tl >> COL_BITS, tl & COL_MASK], tail_v,
                               mask=tm)

        start_out(r)

        if r + 2 < ROUNDS:
            if r >= 1:
                wait_out(r - 1)
            start_in(r + 2)

    for r in range(max(ROUNDS - NBUF, 0), ROUNDS):
        wait_out(r)


_sc_call = functools.partial(
    pl.kernel,
    out_type=jax.ShapeDtypeStruct(SHAPE, jnp.float32),
    mesh=plsc.VectorSubcoreMesh(core_axis_name="c", subcore_axis_name="s"),
    compiler_params=pltpu.CompilerParams(needs_layout_passes=False),
    scratch_types=(
        [pltpu.VMEM((ROWS_PER_CHUNK, SHAPE[1]), jnp.float32)] * NBUF
        + [
            pltpu.VMEM((VB,), jnp.int32),
            pltpu.VMEM((VB,), jnp.float32),
            pltpu.VMEM((16,), jnp.int32),
            pltpu.VMEM((16,), jnp.int32),
            pltpu.VMEM((16,), jnp.int32),
            pltpu.VMEM((16,), jnp.int32),
            pltpu.VMEM((16,), jnp.int32),
            pltpu.VMEM((16,), jnp.int32),
            pltpu.VMEM((16,), jnp.int32),
            pltpu.VMEM((16,), jnp.float32),
        ]
        + [pltpu.SemaphoreType.DMA] * (2 * NBUF)
        + [pltpu.SemaphoreType.DMA]
    ),
)(_sc_body)


def kernel(tensor, values, indices):
    return _sc_call(tensor, values, indices)


# speedup vs baseline: 1.2435x; 1.0158x over previous
import functools

import jax
import jax.numpy as jnp
from jax import lax
from jax.experimental import pallas as pl
from jax.experimental.pallas import tpu as pltpu
from jax.experimental.pallas import tpu_sc as plsc

SHAPE = (4096, 4096)
NUMEL = SHAPE[0] * SHAPE[1]
K = 1677721

NC = 2
NS = 16
NW = NC * NS

CH = 32768
ROWS_PER_CHUNK = CH // SHAPE[1]
NCHUNK = NUMEL // CH
ROUNDS = NCHUNK // NW
NBUF = 3
VB = 4096
NTAIL = K % 8
K_MAIN = K - NTAIL
BS_ITERS = K.bit_length()
COL_BITS = SHAPE[1].bit_length() - 1
COL_MASK = SHAPE[1] - 1


def _sc_body(tensor_hbm, values_hbm, indices_hbm, out_hbm,
             acc0, acc1, acc2, idxb, valb, stv0, stv1, midb, gatb,
             midb2, gatb2, tiv, tvv,
             isem0, isem1, isem2, osem0, osem1, osem2, msem):
    accs = (acc0, acc1, acc2)
    isems = (isem0, isem1, isem2)
    osems = (osem0, osem1, osem2)
    stvs = (stv0, stv1)

    cidx = lax.axis_index("c")
    sidx = lax.axis_index("s")
    wid = sidx * NC + cidx
    lanes = lax.iota(jnp.int32, 16)

    def chunk_row0(r):
        return pl.multiple_of((wid * ROUNDS + r) * ROWS_PER_CHUNK,
                              ROWS_PER_CHUNK)

    def start_in(r):
        pltpu.make_async_copy(
            tensor_hbm.at[pl.ds(chunk_row0(r), ROWS_PER_CHUNK), :],
            accs[r % NBUF], isems[r % NBUF]).start()

    def wait_in(r):
        pltpu.make_async_copy(
            tensor_hbm.at[pl.ds(chunk_row0(r), ROWS_PER_CHUNK), :],
            accs[r % NBUF], isems[r % NBUF]).wait()

    def start_out(r):
        pltpu.make_async_copy(
            accs[r % NBUF],
            out_hbm.at[pl.ds(chunk_row0(r), ROWS_PER_CHUNK), :],
            osems[r % NBUF]).start()

    def wait_out(r):
        pltpu.make_async_copy(
            accs[r % NBUF],
            out_hbm.at[pl.ds(chunk_row0(r), ROWS_PER_CHUNK), :],
            osems[r % NBUF]).wait()

    for r in range(min(2, ROUNDS)):
        start_in(r)

    tiv[...] = jnp.full((16,), -1, jnp.int32)
    tvv[...] = jnp.zeros((16,), jnp.float32)
    if NTAIL:
        tic = pltpu.async_copy(
            indices_hbm.at[pl.ds(K_MAIN, NTAIL)],
            tiv.at[pl.ds(0, NTAIL)], msem)
        tvc = pltpu.async_copy(
            values_hbm.at[pl.ds(K_MAIN, NTAIL)],
            tvv.at[pl.ds(0, NTAIL)], msem)
        tic.wait()
        tvc.wait()
    tail_i = tiv[...]
    tail_v = tvv[...]

    targets1 = (wid * ROUNDS + lanes) * CH
    targets2 = ((wid + 1) * ROUNDS) * CH + jnp.zeros((16,), jnp.int32)

    def bs_body(it, carry):
        lo1, hi1, lo2, hi2 = carry
        a1 = lo1 < hi1
        a2 = lo2 < hi2
        mid1 = lo1 + ((hi1 - lo1) >> 1)
        mid2 = lo2 + ((hi2 - lo2) >> 1)
        midb[...] = jnp.minimum(mid1, K - 1)
        midb2[...] = jnp.minimum(mid2, K - 1)
        c1 = pltpu.async_copy(indices_hbm.at[midb], gatb, msem)
        c2 = pltpu.async_copy(indices_hbm.at[midb2], gatb2, msem)
        c1.wait()
        c2.wait()
        gr1 = gatb[...] < targets1
        gr2 = gatb2[...] < targets2
        return (jnp.where(a1 & gr1, mid1 + 1, lo1),
                jnp.where(a1 & (~gr1), mid1, hi1),
                jnp.where(a2 & gr2, mid2 + 1, lo2),
                jnp.where(a2 & (~gr2), mid2, hi2))

    z16 = jnp.zeros((16,), jnp.int32)
    k16 = jnp.full((16,), K, jnp.int32)
    lo1_v, _, lo2_v, _ = lax.fori_loop(
        0, BS_ITERS, bs_body, (z16, k16, z16, k16))
    stvs[0][...] = lo1_v
    stvs[1][...] = lo2_v

    sv0 = stvs[0][...]
    sv1 = stvs[1][...]

    def bound(i):
        return sv0[i] if i < 16 else sv1[0]

    for r in range(ROUNDS):
        acc = accs[r % NBUF]
        chunk = wid * ROUNDS + r
        lo = pl.multiple_of(chunk * CH, CH)
        p0 = bound(r)
        p1 = bound(r + 1)
        p1e = jnp.minimum(p1, K_MAIN)

        wait_in(r)

        sbase0 = pl.multiple_of(
            jnp.maximum(jnp.minimum(p0 & -8, K_MAIN - VB), 0), 8)
        nb = jnp.maximum((p1e - sbase0 + (VB - 1)) // VB, 0)

        def batch_body(i, carry, _p0=p0, _p1e=p1e, _lo=lo, _sbase0=sbase0,
                       _acc=acc):
            ustart = _sbase0 + i * VB
            sbase = pl.multiple_of(jnp.minimum(ustart, K_MAIN - VB), 8)
            vcp = pltpu.async_copy(values_hbm.at[pl.ds(sbase, VB)], valb,
                                   msem)
            icp = pltpu.async_copy(indices_hbm.at[pl.ds(sbase, VB)], idxb,
                                   msem)
            vcp.wait()
            icp.wait()
            cur = jnp.maximum(_p0, ustart)
            j0 = (cur - sbase) >> 4
            jend = (jnp.minimum(_p1e, sbase + VB) - sbase + 15) >> 4

            def edge(j):
                off = pl.multiple_of(j * 16, 16)
                iv = idxb[pl.ds(off, 16)]
                vv = valb[pl.ds(off, 16)]
                pos = sbase + j * 16 + lanes
                m = (pos >= cur) & (pos < _p1e)
                liv = jnp.where(m, iv - _lo, 0)
                plsc.addupdate_scatter(
                    _acc, [liv >> COL_BITS, liv & COL_MASK], vv, mask=m)

            def mid(j, carry2):
                off = pl.multiple_of(j * 16, 16)
                iv = idxb[pl.ds(off, 16)]
                vv = valb[pl.ds(off, 16)]
                liv = iv - _lo
                plsc.addupdate_scatter(
                    _acc, [liv >> COL_BITS, liv & COL_MASK], vv)
                return carry2

            edge(j0)
            lax.fori_loop(j0 + 1, jend - 1, mid, 0)

            @pl.when(jend - 1 > j0)
            def _():
                edge(jend - 1)

            return carry

        lax.fori_loop(0, nb, batch_body, 0)

        tm = (tail_i >= lo) & (tail_i < lo + CH)
        tl = jnp.where(tm, tail_i - lo, 0)
        plsc.addupdate_scatter(acc, [tl >> COL_BITS, tl & COL_MASK], tail_v,
                               mask=tm)

        start_out(r)

        if r + 2 < ROUNDS:
            if r >= 1:
                wait_out(r - 1)
            start_in(r + 2)

    for r in range(max(ROUNDS - NBUF, 0), ROUNDS):
        wait_out(r)


_sc_call = functools.partial(
    pl.kernel,
    out_type=jax.ShapeDtypeStruct(SHAPE, jnp.float32),
    mesh=plsc.VectorSubcoreMesh(core_axis_name="c", subcore_axis_name="s"),
    compiler_params=pltpu.CompilerParams(needs_layout_passes=False),
    scratch_types=(
        [pltpu.VMEM((ROWS_PER_CHUNK, SHAPE[1]), jnp.float32)] * NBUF
        + [
            pltpu.VMEM((VB,), jnp.int32),
            pltpu.VMEM((VB,), jnp.float32),
            pltpu.VMEM((16,), jnp.int32),
            pltpu.VMEM((16,), jnp.int32),
            pltpu.VMEM((16,), jnp.int32),
            pltpu.VMEM((16,), jnp.int32),
            pltpu.VMEM((16,), jnp.int32),
            pltpu.VMEM((16,), jnp.int32),
            pltpu.VMEM((16,), jnp.int32),
            pltpu.VMEM((16,), jnp.float32),
        ]
        + [pltpu.SemaphoreType.DMA] * (2 * NBUF)
        + [pltpu.SemaphoreType.DMA]
    ),
)(_sc_body)


def kernel(tensor, values, indices):
    return _sc_call(tensor, values, indices)
